# baseline (speedup 1.0000x reference)
import functools

import jax
import jax.numpy as jnp
from jax import lax
from jax.experimental import pallas as pl
from jax.experimental.pallas import tpu as pltpu
from jax.experimental.pallas import tpu_sc as plsc

_IDX = 13
_LANE = 128
_GRP = 4
_NBUF = 4


@functools.lru_cache(maxsize=None)
def _make_sc_gather(V, D, B):
    info = plsc.get_sparse_core_info()
    NC, NS = info.num_cores, info.num_subcores
    NW = NC * NS
    b_per_w = B // NW
    n_chunks = b_per_w // 16
    n_grp = b_per_w // _GRP

    mesh = plsc.VectorSubcoreMesh(core_axis_name="c", subcore_axis_name="s")

    @functools.partial(
        pl.kernel,
        mesh=mesh,
        compiler_params=pltpu.CompilerParams(needs_layout_passes=False),
        out_type=jax.ShapeDtypeStruct((D, B), jnp.float32),
        scratch_types=[
            pltpu.VMEM((b_per_w,), jnp.int32),
            pltpu.VMEM((_NBUF * _GRP, D, _LANE), jnp.float32),
            pltpu.VMEM((D, b_per_w), jnp.float32),
            pltpu.SemaphoreType.DMA,
        ],
    )
    def gather_k(tableT_hbm, idx_hbm, out_hbm, idx_v, slabs_v, rows_v, sem):
        wid = lax.axis_index("s") * NC + lax.axis_index("c")
        base = wid * b_per_w
        pltpu.sync_copy(idx_hbm.at[pl.ds(base, b_per_w)], idx_v)
        iota16 = lax.iota(jnp.int32, 16)

        def fire(g, chunk, lane0):
            slot = (g % _NBUF) * _GRP
            for i in range(_GRP):
                sid = chunk[lane0 + i]
                colbase = pl.multiple_of((sid // _LANE) * _LANE, _LANE)
                pltpu.async_copy(
                    tableT_hbm.at[:, pl.ds(colbase, _LANE)],
                    slabs_v.at[slot + i],
                    sem,
                )

        def process(g, chunk, lane0):
            slot = (g % _NBUF) * _GRP
            for _ in range(_GRP):
                pltpu.make_async_copy(
                    tableT_hbm.at[:, pl.ds(0, _LANE)],
                    slabs_v.at[0],
                    sem,
                ).wait()
            row0 = g * _GRP
            for i in range(_GRP):
                sid = chunk[lane0 + i]
                col = jnp.full((16,), sid % _LANE, jnp.int32)
                posv = jnp.full((16,), row0 + i, jnp.int32)
                slotv = jnp.full((16,), slot + i, jnp.int32)
                for h in range(D // 16):
                    vals = plsc.load_gather(
                        slabs_v, [slotv, iota16 + 16 * h, col]
                    )
                    plsc.store_scatter(
                        rows_v, [iota16 + 16 * h, posv], vals
                    )

        c0 = idx_v[pl.ds(0, 16)]
        fire(0, c0, 0)
        fire(1, c0, 4)
        fire(2, c0, 8)

        def body(c, carry):
            cur = idx_v[pl.ds(c * 16, 16)]
            nxt = idx_v[pl.ds((c + 1) * 16, 16)]
            for sub in range(4):
                g = c * 4 + sub
                fsub = sub + 3
                if fsub < 4:
                    fire(g + 3, cur, 4 * fsub)
                else:
                    fire(g + 3, nxt, 4 * (fsub - 4))
                process(g, cur, 4 * sub)
            return carry

        lax.fori_loop(0, n_chunks - 1, body, 0)
        clast = idx_v[pl.ds((n_chunks - 1) * 16, 16)]
        for sub in range(4):
            g = (n_chunks - 1) * 4 + sub
            if sub + 3 < 4:
                fire(g + 3, clast, 4 * (sub + 3))
            process(g, clast, 4 * sub)

        pltpu.sync_copy(rows_v, out_hbm.at[:, pl.ds(base, b_per_w)])

    return gather_k


def _mlp_body(xT_ref, eT_ref, w1dT_ref, w1eT_ref, b1_ref, w2T_ref, b2_ref, oT_ref):
    h = jnp.dot(w1dT_ref[...], xT_ref[...], preferred_element_type=jnp.float32)
    h = h + jnp.dot(w1eT_ref[...], eT_ref[...], preferred_element_type=jnp.float32)
    h = jnp.maximum(h + b1_ref[...], 0.0)
    o = jnp.dot(w2T_ref[...], h, preferred_element_type=jnp.float32) + b2_ref[...]
    oT_ref[...] = jnp.maximum(o, 0.0)


def _mlp(xT, embT, W1dT, W1eT, b1c, W2T, b2c, block_b=8192):
    F, B = xT.shape
    OUT, HID = W2T.shape
    D = embT.shape[0]
    grid = (B // block_b,)
    return pl.pallas_call(
        _mlp_body,
        grid=grid,
        in_specs=[
            pl.BlockSpec((F, block_b), lambda i: (0, i)),
            pl.BlockSpec((D, block_b), lambda i: (0, i)),
            pl.BlockSpec((HID, F), lambda i: (0, 0)),
            pl.BlockSpec((HID, D), lambda i: (0, 0)),
            pl.BlockSpec((HID, 1), lambda i: (0, 0)),
            pl.BlockSpec((OUT, HID), lambda i: (0, 0)),
            pl.BlockSpec((OUT, 1), lambda i: (0, 0)),
        ],
        out_specs=pl.BlockSpec((OUT, block_b), lambda i: (0, i)),
        out_shape=jax.ShapeDtypeStruct((OUT, B), jnp.float32),
    )(xT, embT, W1dT, W1eT, b1c, W2T, b2c)


def kernel(inputs, table, W1, b1, W2, b2):
    B, F = inputs.shape
    V, D = table.shape
    HID = W1.shape[1]
    tableT = table.T
    inputsT = inputs.T
    idx = inputsT[_IDX].astype(jnp.int32)
    embT = _make_sc_gather(V, D, B)(tableT, idx)
    W1T = W1.T
    W1dT = jnp.concatenate(
        [W1T[:, :_IDX], jnp.zeros((HID, 1), W1.dtype), W1T[:, _IDX : F - 1]],
        axis=1,
    )
    W1eT = W1T[:, F - 1 :]
    outT = _mlp(
        inputsT, embT, W1dT, W1eT, b1.reshape(-1, 1), W2.T, b2.reshape(-1, 1)
    )
    return outT.T


# speedup vs baseline: 1.0001x; 1.0001x over previous
import functools

import jax
import jax.numpy as jnp
from jax import lax
from jax.experimental import pallas as pl
from jax.experimental.pallas import tpu as pltpu
from jax.experimental.pallas import tpu_sc as plsc

_IDX = 13
_LANE = 128
_GRP = 4
_NBUF = 4


@functools.lru_cache(maxsize=None)
def _make_sc_gather(V, D, B):
    info = plsc.get_sparse_core_info()
    NC, NS = info.num_cores, info.num_subcores
    NW = NC * NS
    b_per_w = B // NW
    n_chunks = b_per_w // 16

    mesh = plsc.VectorSubcoreMesh(core_axis_name="c", subcore_axis_name="s")

    @functools.partial(
        pl.kernel,
        mesh=mesh,
        compiler_params=pltpu.CompilerParams(needs_layout_passes=False),
        out_type=jax.ShapeDtypeStruct((D, B), jnp.float32),
        scratch_types=[
            pltpu.VMEM((b_per_w,), jnp.int32),
            pltpu.VMEM((_NBUF * _GRP, D, _LANE), jnp.float32),
            pltpu.VMEM((D, b_per_w), jnp.float32),
            pltpu.SemaphoreType.DMA,
        ],
    )
    def gather_k(tableT_hbm, idx_hbm, out_hbm, idx_v, slabs_v, rows_v, sem):
        wid = lax.axis_index("s") * NC + lax.axis_index("c")
        base = wid * b_per_w
        pltpu.sync_copy(idx_hbm.at[pl.ds(base, b_per_w)], idx_v)
        iota16 = lax.iota(jnp.int32, 16)

        def fire(g, chunk, lane0):
            slot = (g % _NBUF) * _GRP
            for i in range(_GRP):
                sid = chunk[lane0 + i]
                colbase = pl.multiple_of((sid // _LANE) * _LANE, _LANE)
                pltpu.async_copy(
                    tableT_hbm.at[:, pl.ds(colbase, _LANE)],
                    slabs_v.at[slot + i],
                    sem,
                )

        def process(g, chunk, lane0):
            slot = (g % _NBUF) * _GRP
            for _ in range(_GRP):
                pltpu.make_async_copy(
                    tableT_hbm.at[:, pl.ds(0, _LANE)],
                    slabs_v.at[0],
                    sem,
                ).wait()
            row0 = g * _GRP
            for i in range(_GRP):
                sid = chunk[lane0 + i]
                col = jnp.full((16,), sid % _LANE, jnp.int32)
                posv = jnp.full((16,), row0 + i, jnp.int32)
                slotv = jnp.full((16,), slot + i, jnp.int32)
                for h in range(D // 16):
                    vals = plsc.load_gather(
                        slabs_v, [slotv, iota16 + 16 * h, col]
                    )
                    plsc.store_scatter(
                        rows_v, [iota16 + 16 * h, posv], vals
                    )

        c0 = idx_v[pl.ds(0, 16)]
        fire(0, c0, 0)
        fire(1, c0, 4)
        fire(2, c0, 8)

        def body(c, carry):
            cur = idx_v[pl.ds(c * 16, 16)]
            nxt = idx_v[pl.ds((c + 1) * 16, 16)]
            for sub in range(4):
                g = c * 4 + sub
                fsub = sub + 3
                if fsub < 4:
                    fire(g + 3, cur, 4 * fsub)
                else:
                    fire(g + 3, nxt, 4 * (fsub - 4))
                process(g, cur, 4 * sub)
            return carry

        lax.fori_loop(0, n_chunks - 1, body, 0)
        clast = idx_v[pl.ds((n_chunks - 1) * 16, 16)]
        for sub in range(4):
            g = (n_chunks - 1) * 4 + sub
            if sub + 3 < 4:
                fire(g + 3, clast, 4 * (sub + 3))
            process(g, clast, 4 * sub)

        pltpu.sync_copy(rows_v, out_hbm.at[:, pl.ds(base, b_per_w)])

    return gather_k


def _mlp_body(xT_ref, eT_ref, w1dT_ref, w1eT_ref, b1_ref, w2T_ref, b2_ref, oT_ref):
    h = jnp.dot(w1dT_ref[...], xT_ref[...], preferred_element_type=jnp.float32)
    h = h + jnp.dot(w1eT_ref[...], eT_ref[...], preferred_element_type=jnp.float32)
    h = jnp.maximum(h + b1_ref[...], 0.0)
    o = jnp.dot(w2T_ref[...], h, preferred_element_type=jnp.float32) + b2_ref[...]
    oT_ref[...] = jnp.maximum(o, 0.0)


def _mlp(xT, embT, W1dT, W1eT, b1c, W2T, b2c, block_b=8192):
    F, B = xT.shape
    OUT, HID = W2T.shape
    D = embT.shape[0]
    grid = (B // block_b,)
    return pl.pallas_call(
        _mlp_body,
        grid=grid,
        in_specs=[
            pl.BlockSpec((F, block_b), lambda i: (0, i)),
            pl.BlockSpec((D, block_b), lambda i: (0, i)),
            pl.BlockSpec((HID, F), lambda i: (0, 0)),
            pl.BlockSpec((HID, D), lambda i: (0, 0)),
            pl.BlockSpec((HID, 1), lambda i: (0, 0)),
            pl.BlockSpec((OUT, HID), lambda i: (0, 0)),
            pl.BlockSpec((OUT, 1), lambda i: (0, 0)),
        ],
        out_specs=pl.BlockSpec((OUT, block_b), lambda i: (0, i)),
        out_shape=jax.ShapeDtypeStruct((OUT, B), jnp.float32),
    )(xT, embT, W1dT, W1eT, b1c, W2T, b2c)


def kernel(inputs, table, W1, b1, W2, b2):
    B, F = inputs.shape
    V, D = table.shape
    HID = W1.shape[1]
    tableT = table.T
    inputsT = inputs.T
    idx = inputsT[_IDX].astype(jnp.int32)
    embT = _make_sc_gather(V, D, B)(tableT, idx)
    W1T = W1.T
    W1dT = jnp.concatenate(
        [W1T[:, :_IDX], jnp.zeros((HID, 1), W1.dtype), W1T[:, _IDX : F - 1]],
        axis=1,
    )
    W1eT = W1T[:, F - 1 :]
    outT = _mlp(
        inputsT, embT, W1dT, W1eT, b1.reshape(-1, 1), W2.T, b2.reshape(-1, 1)
    )
    return outT.T
